# baseline (speedup 1.0000x reference)
import functools

import jax
import jax.numpy as jnp
from jax.experimental import pallas as pl
from jax.experimental.pallas import tpu as pltpu


def _bow_kernel(feat_ref, maskf_ref, cent_ref, hist_ref, cnorm_ref, *,
                blk, num_bags):
    b = pl.program_id(0)
    i = pl.program_id(1)
    cent = cent_ref[...]

    @pl.when(jnp.logical_and(b == 0, i == 0))
    def _cnorm():
        cnorm_ref[...] = jnp.sum(cent * cent, axis=1)[None, :]

    feat = feat_ref[0]
    scores = jax.lax.dot_general(
        feat, cent, (((1,), (1,)), ((), ())),
        preferred_element_type=jnp.float32)
    rnorm = jnp.sum(feat * feat, axis=1, keepdims=True)
    dists = rnorm - 2.0 * scores + cnorm_ref[...]
    nearest = jnp.argmin(dists, axis=1)
    valid = 1.0 - maskf_ref[0, 0]
    eq = (nearest[:, None]
          == jax.lax.broadcasted_iota(jnp.int32, (blk, num_bags), 1))
    onehot = jnp.where(eq, valid[:, None], 0.0)
    contrib = jax.lax.dot_general(
        jnp.ones((1, blk), jnp.float32), onehot,
        (((1,), (0,)), ((), ())),
        preferred_element_type=jnp.float32)

    @pl.when(i == 0)
    def _init():
        hist_ref[...] = contrib[None]

    @pl.when(i != 0)
    def _acc():
        hist_ref[...] += contrib[None]


def kernel(features, mask, centroids):
    nb, nc, d = features.shape
    num_bags = centroids.shape[0]
    blk = 512
    num_blk = nc // blk
    maskf = mask.astype(jnp.float32).reshape(nb * num_blk, 1, blk)

    grid = (nb, num_blk)
    hist = pl.pallas_call(
        functools.partial(_bow_kernel, blk=blk, num_bags=num_bags),
        grid=grid,
        in_specs=[
            pl.BlockSpec((1, blk, d), lambda b, i: (b, i, 0)),
            pl.BlockSpec((1, 1, blk), lambda b, i, nbk=num_blk: (b * nbk + i, 0, 0)),
            pl.BlockSpec((num_bags, d), lambda b, i: (0, 0)),
        ],
        out_specs=pl.BlockSpec((1, 1, num_bags), lambda b, i: (b, 0, 0)),
        out_shape=jax.ShapeDtypeStruct((nb, 1, num_bags), jnp.float32),
        scratch_shapes=[pltpu.VMEM((1, num_bags), jnp.float32)],
        compiler_params=pltpu.CompilerParams(
            dimension_semantics=("arbitrary", "arbitrary")),
    )(features, maskf, centroids)
    return hist.reshape(nb, num_bags)


# speedup vs baseline: 1.1251x; 1.1251x over previous
import functools

import jax
import jax.numpy as jnp
from jax.experimental import pallas as pl
from jax.experimental.pallas import tpu as pltpu


def _bow_kernel(feat_ref, maskf_ref, cent_ref, hist_ref, cnorm_ref, *,
                blk, num_bags):
    b = pl.program_id(0)
    i = pl.program_id(1)
    cent = cent_ref[...]

    @pl.when(jnp.logical_and(b == 0, i == 0))
    def _cnorm():
        cnorm_ref[...] = jnp.sum(cent * cent, axis=1)[None, :]

    feat = feat_ref[0]
    scores = jax.lax.dot_general(
        feat, cent, (((1,), (1,)), ((), ())),
        preferred_element_type=jnp.float32)
    rnorm = jnp.sum(feat * feat, axis=1, keepdims=True)
    dists = rnorm - 2.0 * scores + cnorm_ref[...]
    nearest = jnp.argmin(dists, axis=1)
    valid = 1.0 - maskf_ref[0, 0]
    eq = (nearest[:, None]
          == jax.lax.broadcasted_iota(jnp.int32, (blk, num_bags), 1))
    onehot = jnp.where(eq, valid[:, None], 0.0)
    contrib = jnp.sum(onehot, axis=0)

    @pl.when(i == 0)
    def _init():
        hist_ref[...] = contrib[None, None, :]

    @pl.when(i != 0)
    def _acc():
        hist_ref[...] += contrib[None, None, :]


def kernel(features, mask, centroids):
    nb, nc, d = features.shape
    num_bags = centroids.shape[0]
    blk = 512
    num_blk = nc // blk
    maskf = mask.astype(jnp.float32).reshape(nb * num_blk, 1, blk)

    grid = (nb, num_blk)
    hist = pl.pallas_call(
        functools.partial(_bow_kernel, blk=blk, num_bags=num_bags),
        grid=grid,
        in_specs=[
            pl.BlockSpec((1, blk, d), lambda b, i: (b, i, 0)),
            pl.BlockSpec((1, 1, blk), lambda b, i, nbk=num_blk: (b * nbk + i, 0, 0)),
            pl.BlockSpec((num_bags, d), lambda b, i: (0, 0)),
        ],
        out_specs=pl.BlockSpec((1, 1, num_bags), lambda b, i: (b, 0, 0)),
        out_shape=jax.ShapeDtypeStruct((nb, 1, num_bags), jnp.float32),
        scratch_shapes=[pltpu.VMEM((1, num_bags), jnp.float32)],
        compiler_params=pltpu.CompilerParams(
            dimension_semantics=("arbitrary", "arbitrary")),
    )(features, maskf, centroids)
    return hist.reshape(nb, num_bags)


# speedup vs baseline: 1.2231x; 1.0870x over previous
import functools

import jax
import jax.numpy as jnp
from jax import lax
from jax.experimental import pallas as pl
from jax.experimental.pallas import tpu as pltpu
from jax.experimental.pallas import tpu_sc as plsc


def _assign_kernel(feat_ref, maskf_ref, cent_ref, near_ref, cnorm_ref, *,
                   blk, num_bags):
    b = pl.program_id(0)
    i = pl.program_id(1)
    cent = cent_ref[...]

    @pl.when(jnp.logical_and(b == 0, i == 0))
    def _cnorm():
        cnorm_ref[...] = jnp.sum(cent * cent, axis=1, keepdims=True)

    feat = feat_ref[0]
    scores = jax.lax.dot_general(
        cent, feat, (((1,), (1,)), ((), ())),
        preferred_element_type=jnp.float32)
    dists = cnorm_ref[...] - 2.0 * scores
    nearest = jnp.argmin(dists, axis=0).astype(jnp.int32)
    masked = maskf_ref[0, 0] > 0.5
    nearest = jnp.where(masked, num_bags, nearest)
    near_ref[...] = nearest[None, None, :]


def _histogram_sc(near_hbm, out_hbm, idx_v, hrows, hv, pv, shared):
    c = lax.axis_index("c")
    s = lax.axis_index("s")
    b_local = s % 8
    half = s // 8
    batch = c * 8 + b_local
    nbins = hrows.shape[1]
    items = idx_v.shape[0]
    base = batch * (2 * items) + half * items

    pltpu.sync_copy(near_hbm.at[pl.ds(base, items)], idx_v)

    zeros16 = jnp.zeros((16,), jnp.float32)
    ones16 = jnp.ones((16,), jnp.float32)
    iota16 = lax.iota(jnp.int32, 16)

    def _zero_body(k, _):
        for r in range(16):
            hrows[r, pl.ds(k * 16, 16)] = zeros16
        return 0

    lax.fori_loop(0, nbins // 16, _zero_body, 0)

    def _acc_body(j, _):
        idxs = idx_v[pl.ds(j * 16, 16)]
        plsc.addupdate_scatter(hrows, [iota16, idxs], ones16)
        return 0

    lax.fori_loop(0, items // 16, _acc_body, 0)

    nred = hv.shape[0] // 16
    for k in range(nred):
        acc = hrows[0, pl.ds(k * 16, 16)]
        for r in range(1, 16):
            acc = acc + hrows[r, pl.ds(k * 16, 16)]
        hv[pl.ds(k * 16, 16)] = acc

    @pl.when(half == 1)
    def _publish():
        pltpu.sync_copy(hv, shared.at[b_local])

    plsc.subcore_barrier()

    @pl.when(half == 0)
    def _merge():
        pltpu.sync_copy(shared.at[b_local], pv)
        for k in range(nred):
            hv[pl.ds(k * 16, 16)] = (hv[pl.ds(k * 16, 16)]
                                     + pv[pl.ds(k * 16, 16)])
        pltpu.sync_copy(hv, out_hbm.at[batch])


def kernel(features, mask, centroids):
    nb, nc, d = features.shape
    num_bags = centroids.shape[0]
    blk = 512
    num_blk = nc // blk
    maskf = mask.astype(jnp.float32).reshape(nb * num_blk, 1, blk)

    nearest = pl.pallas_call(
        functools.partial(_assign_kernel, blk=blk, num_bags=num_bags),
        grid=(nb, num_blk),
        in_specs=[
            pl.BlockSpec((1, blk, d), lambda b, i: (b, i, 0)),
            pl.BlockSpec((1, 1, blk), lambda b, i, nbk=num_blk: (b * nbk + i, 0, 0)),
            pl.BlockSpec((num_bags, d), lambda b, i: (0, 0)),
        ],
        out_specs=pl.BlockSpec((1, 1, blk), lambda b, i, nbk=num_blk: (b * nbk + i, 0, 0)),
        out_shape=jax.ShapeDtypeStruct((nb * num_blk, 1, blk), jnp.int32),
        scratch_shapes=[pltpu.VMEM((num_bags, 1), jnp.float32)],
        compiler_params=pltpu.CompilerParams(
            dimension_semantics=("arbitrary", "arbitrary")),
    )(features, maskf, centroids)

    flat_nearest = nearest.reshape(nb * nc)
    items_per_worker = (nb * nc) // 32

    hist = pl.kernel(
        _histogram_sc,
        mesh=plsc.VectorSubcoreMesh(core_axis_name="c", subcore_axis_name="s"),
        compiler_params=pltpu.CompilerParams(use_tc_tiling_on_sc=False,
                                             needs_layout_passes=False),
        out_type=jax.ShapeDtypeStruct((nb, num_bags), jnp.float32),
        scratch_types=[
            pltpu.VMEM((items_per_worker,), jnp.int32),
            pltpu.VMEM((16, num_bags + 16), jnp.float32),
            pltpu.VMEM((num_bags,), jnp.float32),
            pltpu.VMEM((num_bags,), jnp.float32),
            pltpu.VMEM_SHARED((8, num_bags), jnp.float32),
        ],
    )(flat_nearest)
    return hist


# speedup vs baseline: 1.2555x; 1.0265x over previous
import functools

import jax
import jax.numpy as jnp
from jax import lax
from jax.experimental import pallas as pl
from jax.experimental.pallas import tpu as pltpu
from jax.experimental.pallas import tpu_sc as plsc


def _assign_kernel(feat_ref, maskf_ref, cent_ref, near_ref, cnorm_ref,
                   sc_ref, *, blk, num_bags):
    k = pl.program_id(0)
    cent = cent_ref[...]

    @pl.when(k == 0)
    def _cnorm():
        cnorm_ref[...] = jnp.sum(cent * cent, axis=1, keepdims=True)

    feat = feat_ref[0]
    scores = jax.lax.dot_general(
        cent, feat, (((1,), (1,)), ((), ())),
        preferred_element_type=jnp.float32)
    dists = cnorm_ref[...] - 2.0 * sc_ref[...]
    nearest = jnp.argmin(dists, axis=0).astype(jnp.int32)
    masked = maskf_ref[0, 0] > 0.5
    nearest = jnp.where(masked, num_bags, nearest)
    near_ref[...] = nearest[None, None, :]
    sc_ref[...] = scores


def _histogram_sc(near_hbm, out_hbm, idx_v, hrows, hv, pv, shared):
    c = lax.axis_index("c")
    s = lax.axis_index("s")
    b_local = s % 8
    half = s // 8
    batch = c * 8 + b_local
    nbins = hrows.shape[1]
    items = idx_v.shape[0]
    base = batch * (2 * items) + half * items

    pltpu.sync_copy(near_hbm.at[pl.ds(base, items)], idx_v)

    zeros16 = jnp.zeros((16,), jnp.float32)
    ones16 = jnp.ones((16,), jnp.float32)
    iota16 = lax.iota(jnp.int32, 16)

    def _zero_body(k, _):
        for r in range(16):
            hrows[r, pl.ds(k * 16, 16)] = zeros16
        return 0

    lax.fori_loop(0, nbins // 16, _zero_body, 0)

    def _acc_body(j, _):
        idxs = idx_v[pl.ds(j * 16, 16)]
        plsc.addupdate_scatter(hrows, [iota16, idxs], ones16)
        return 0

    lax.fori_loop(0, items // 16, _acc_body, 0)

    nred = hv.shape[0] // 16
    for k in range(nred):
        acc = hrows[0, pl.ds(k * 16, 16)]
        for r in range(1, 16):
            acc = acc + hrows[r, pl.ds(k * 16, 16)]
        hv[pl.ds(k * 16, 16)] = acc

    @pl.when(half == 1)
    def _publish():
        pltpu.sync_copy(hv, shared.at[b_local])

    plsc.subcore_barrier()

    @pl.when(half == 0)
    def _merge():
        pltpu.sync_copy(shared.at[b_local], pv)
        for k in range(nred):
            hv[pl.ds(k * 16, 16)] = (hv[pl.ds(k * 16, 16)]
                                     + pv[pl.ds(k * 16, 16)])
        pltpu.sync_copy(hv, out_hbm.at[batch])


def kernel(features, mask, centroids):
    nb, nc, d = features.shape
    num_bags = centroids.shape[0]
    blk = 512
    num_blk = nc // blk
    maskf = mask.astype(jnp.float32).reshape(nb * num_blk, 1, blk)

    nblocks = nb * num_blk
    feats3 = features.reshape(nblocks, blk, d)
    nearest = pl.pallas_call(
        functools.partial(_assign_kernel, blk=blk, num_bags=num_bags),
        grid=(nblocks + 1,),
        in_specs=[
            pl.BlockSpec((1, blk, d),
                         lambda k, n=nblocks: (jnp.minimum(k, n - 1), 0, 0)),
            pl.BlockSpec((1, 1, blk),
                         lambda k: (jnp.maximum(k - 1, 0), 0, 0)),
            pl.BlockSpec((num_bags, d), lambda k: (0, 0)),
        ],
        out_specs=pl.BlockSpec((1, 1, blk),
                               lambda k: (jnp.maximum(k - 1, 0), 0, 0)),
        out_shape=jax.ShapeDtypeStruct((nblocks, 1, blk), jnp.int32),
        scratch_shapes=[pltpu.VMEM((num_bags, 1), jnp.float32),
                        pltpu.VMEM((num_bags, blk), jnp.float32)],
        compiler_params=pltpu.CompilerParams(
            dimension_semantics=("arbitrary",)),
    )(feats3, maskf, centroids)

    flat_nearest = nearest.reshape(nb * nc)
    items_per_worker = (nb * nc) // 32

    hist = pl.kernel(
        _histogram_sc,
        mesh=plsc.VectorSubcoreMesh(core_axis_name="c", subcore_axis_name="s"),
        compiler_params=pltpu.CompilerParams(use_tc_tiling_on_sc=False,
                                             needs_layout_passes=False),
        out_type=jax.ShapeDtypeStruct((nb, num_bags), jnp.float32),
        scratch_types=[
            pltpu.VMEM((items_per_worker,), jnp.int32),
            pltpu.VMEM((16, num_bags + 16), jnp.float32),
            pltpu.VMEM((num_bags,), jnp.float32),
            pltpu.VMEM((num_bags,), jnp.float32),
            pltpu.VMEM_SHARED((8, num_bags), jnp.float32),
        ],
    )(flat_nearest)
    return hist


# speedup vs baseline: 1.2858x; 1.0242x over previous
import functools

import jax
import jax.numpy as jnp
from jax import lax
from jax.experimental import pallas as pl
from jax.experimental.pallas import tpu as pltpu
from jax.experimental.pallas import tpu_sc as plsc


def _assign_kernel(feat_ref, maskf_ref, cent_ref, near_ref, hcn_ref, *,
                   blk, num_bags, strip):
    k = pl.program_id(0)

    @pl.when(k == 0)
    def _hcn():
        cent = cent_ref[...]
        hcn_ref[...] = 0.5 * jnp.sum(cent * cent, axis=1, keepdims=True)

    feat = feat_ref[0]
    nstrips = num_bags // strip
    slabs = strip // 8
    mv = jnp.full((8, blk), -jnp.inf, jnp.float32)
    iv = jnp.zeros((8, blk), jnp.int32)
    u_iota = jax.lax.broadcasted_iota(jnp.int32, (8, blk), 0)
    for s in range(nstrips):
        cs = cent_ref[s * strip:(s + 1) * strip, :]
        sc = jax.lax.dot_general(
            cs, feat, (((1,), (1,)), ((), ())),
            preferred_element_type=jnp.float32)
        d3 = (sc - hcn_ref[s * strip:(s + 1) * strip, :]
              ).reshape(slabs, 8, blk)
        m = jnp.max(d3, axis=0)
        a = jnp.argmax(d3, axis=0).astype(jnp.int32)
        cand = (s * strip + u_iota) + a * 8
        better = m > mv
        mv = jnp.where(better, m, mv)
        iv = jnp.where(better, cand, iv)
    mm = jnp.max(mv, axis=0)
    nearest = jnp.min(jnp.where(mv == mm[None, :], iv, 2 * num_bags),
                      axis=0).astype(jnp.int32)
    masked = maskf_ref[0, 0] > 0.5
    nearest = jnp.where(masked, num_bags, nearest)
    near_ref[...] = nearest[None, None, :]


def _histogram_sc(near_hbm, out_hbm, idx_v, hrows, hv, pv, shared):
    c = lax.axis_index("c")
    s = lax.axis_index("s")
    b_local = s % 8
    half = s // 8
    batch = c * 8 + b_local
    nbins = hrows.shape[1]
    items = idx_v.shape[0]
    base = batch * (2 * items) + half * items

    pltpu.sync_copy(near_hbm.at[pl.ds(base, items)], idx_v)

    zeros16 = jnp.zeros((16,), jnp.float32)
    ones16 = jnp.ones((16,), jnp.float32)
    iota16 = lax.iota(jnp.int32, 16)

    def _zero_body(k, _):
        for r in range(16):
            hrows[r, pl.ds(k * 16, 16)] = zeros16
        return 0

    lax.fori_loop(0, nbins // 16, _zero_body, 0)

    def _acc_body(j, _):
        idxs = idx_v[pl.ds(j * 16, 16)]
        plsc.addupdate_scatter(hrows, [iota16, idxs], ones16)
        return 0

    lax.fori_loop(0, items // 16, _acc_body, 0)

    nred = hv.shape[0] // 16
    for k in range(nred):
        acc = hrows[0, pl.ds(k * 16, 16)]
        for r in range(1, 16):
            acc = acc + hrows[r, pl.ds(k * 16, 16)]
        hv[pl.ds(k * 16, 16)] = acc

    @pl.when(half == 1)
    def _publish():
        pltpu.sync_copy(hv, shared.at[b_local])

    plsc.subcore_barrier()

    @pl.when(half == 0)
    def _merge():
        pltpu.sync_copy(shared.at[b_local], pv)
        for k in range(nred):
            hv[pl.ds(k * 16, 16)] = (hv[pl.ds(k * 16, 16)]
                                     + pv[pl.ds(k * 16, 16)])
        pltpu.sync_copy(hv, out_hbm.at[batch])


def kernel(features, mask, centroids):
    nb, nc, d = features.shape
    num_bags = centroids.shape[0]
    blk = 512
    num_blk = nc // blk
    maskf = mask.astype(jnp.float32).reshape(nb * num_blk, 1, blk)

    nblocks = nb * num_blk
    feats3 = features.reshape(nblocks, blk, d)
    nearest = pl.pallas_call(
        functools.partial(_assign_kernel, blk=blk, num_bags=num_bags,
                          strip=256),
        grid=(nblocks,),
        in_specs=[
            pl.BlockSpec((1, blk, d), lambda k: (k, 0, 0)),
            pl.BlockSpec((1, 1, blk), lambda k: (k, 0, 0)),
            pl.BlockSpec((num_bags, d), lambda k: (0, 0)),
        ],
        out_specs=pl.BlockSpec((1, 1, blk), lambda k: (k, 0, 0)),
        out_shape=jax.ShapeDtypeStruct((nblocks, 1, blk), jnp.int32),
        scratch_shapes=[pltpu.VMEM((num_bags, 1), jnp.float32)],
        compiler_params=pltpu.CompilerParams(
            dimension_semantics=("arbitrary",)),
    )(feats3, maskf, centroids)

    flat_nearest = nearest.reshape(nb * nc)
    items_per_worker = (nb * nc) // 32

    hist = pl.kernel(
        _histogram_sc,
        mesh=plsc.VectorSubcoreMesh(core_axis_name="c", subcore_axis_name="s"),
        compiler_params=pltpu.CompilerParams(use_tc_tiling_on_sc=False,
                                             needs_layout_passes=False),
        out_type=jax.ShapeDtypeStruct((nb, num_bags), jnp.float32),
        scratch_types=[
            pltpu.VMEM((items_per_worker,), jnp.int32),
            pltpu.VMEM((16, num_bags + 16), jnp.float32),
            pltpu.VMEM((num_bags,), jnp.float32),
            pltpu.VMEM((num_bags,), jnp.float32),
            pltpu.VMEM_SHARED((8, num_bags), jnp.float32),
        ],
    )(flat_nearest)
    return hist


# speedup vs baseline: 1.6580x; 1.2894x over previous
import functools

import jax
import jax.numpy as jnp
from jax import lax
from jax.experimental import pallas as pl
from jax.experimental.pallas import tpu as pltpu
from jax.experimental.pallas import tpu_sc as plsc


def _assign_kernel(feat_ref, maskf_ref, cent_ref, near_ref, hcn_ref, *,
                   blk, num_bags, strip):
    k = pl.program_id(0)

    @pl.when(k == 0)
    def _hcn():
        cent = cent_ref[...]
        hcn_ref[...] = 0.5 * jnp.sum(cent * cent, axis=1, keepdims=True)

    feat = feat_ref[0]
    nstrips = num_bags // strip
    slabs = strip // 8
    mv = jnp.full((8, blk), -jnp.inf, jnp.float32)
    iv = jnp.zeros((8, blk), jnp.int32)
    u_iota = jax.lax.broadcasted_iota(jnp.int32, (8, blk), 0)
    for s in range(nstrips):
        cs = cent_ref[s * strip:(s + 1) * strip, :]
        sc = jax.lax.dot_general(
            cs, feat, (((1,), (1,)), ((), ())),
            preferred_element_type=jnp.float32)
        d3 = (sc - hcn_ref[s * strip:(s + 1) * strip, :]
              ).reshape(slabs, 8, blk)
        m = jnp.max(d3, axis=0)
        a = jnp.argmax(d3, axis=0).astype(jnp.int32)
        cand = (s * strip + u_iota) + a * 8
        better = m > mv
        mv = jnp.where(better, m, mv)
        iv = jnp.where(better, cand, iv)
    mm = jnp.max(mv, axis=0)
    nearest = jnp.min(jnp.where(mv == mm[None, :], iv, 2 * num_bags),
                      axis=0).astype(jnp.int32)
    masked = maskf_ref[0, 0] > 0.5
    nearest = jnp.where(masked, num_bags, nearest)
    near_ref[...] = nearest[None, None, :]


def _histogram_sc(near_hbm, out_hbm, idx_v, hrows, hv, pv, shared):
    c = lax.axis_index("c")
    s = lax.axis_index("s")
    b_local = s % 8
    half = s // 8
    batch = c * 8 + b_local
    nbins = hrows.shape[1]
    items = idx_v.shape[0]
    base = batch * (2 * items) + half * items

    pltpu.sync_copy(near_hbm.at[pl.ds(base, items)], idx_v)

    zeros16 = jnp.zeros((16,), jnp.float32)
    ones16 = jnp.ones((16,), jnp.float32)
    iota16 = lax.iota(jnp.int32, 16)

    def _zero_body(k, _):
        for r in range(16):
            hrows[r, pl.ds(k * 16, 16)] = zeros16
        return 0

    lax.fori_loop(0, nbins // 16, _zero_body, 0)

    def _acc_body(j, _):
        idxs = idx_v[pl.ds(j * 16, 16)]
        plsc.addupdate_scatter(hrows, [iota16, idxs], ones16)
        return 0

    lax.fori_loop(0, items // 16, _acc_body, 0)

    nred = hv.shape[0] // 16
    for k in range(nred):
        acc = hrows[0, pl.ds(k * 16, 16)]
        for r in range(1, 16):
            acc = acc + hrows[r, pl.ds(k * 16, 16)]
        hv[pl.ds(k * 16, 16)] = acc

    @pl.when(half == 1)
    def _publish():
        pltpu.sync_copy(hv, shared.at[b_local])

    plsc.subcore_barrier()

    @pl.when(half == 0)
    def _merge():
        pltpu.sync_copy(shared.at[b_local], pv)
        for k in range(nred):
            hv[pl.ds(k * 16, 16)] = (hv[pl.ds(k * 16, 16)]
                                     + pv[pl.ds(k * 16, 16)])
        pltpu.sync_copy(hv, out_hbm.at[batch])


def kernel(features, mask, centroids):
    nb, nc, d = features.shape
    num_bags = centroids.shape[0]
    blk = 1024
    num_blk = nc // blk
    maskf = mask.astype(jnp.float32).reshape(nb * num_blk, 1, blk)

    nblocks = nb * num_blk
    feats3 = features.reshape(nblocks, blk, d)
    nearest = pl.pallas_call(
        functools.partial(_assign_kernel, blk=blk, num_bags=num_bags,
                          strip=256),
        grid=(nblocks,),
        in_specs=[
            pl.BlockSpec((1, blk, d), lambda k: (k, 0, 0)),
            pl.BlockSpec((1, 1, blk), lambda k: (k, 0, 0)),
            pl.BlockSpec((num_bags, d), lambda k: (0, 0)),
        ],
        out_specs=pl.BlockSpec((1, 1, blk), lambda k: (k, 0, 0)),
        out_shape=jax.ShapeDtypeStruct((nblocks, 1, blk), jnp.int32),
        scratch_shapes=[pltpu.VMEM((num_bags, 1), jnp.float32)],
        compiler_params=pltpu.CompilerParams(
            dimension_semantics=("arbitrary",)),
    )(feats3, maskf, centroids)

    flat_nearest = nearest.reshape(nb * nc)
    items_per_worker = (nb * nc) // 32

    hist = pl.kernel(
        _histogram_sc,
        mesh=plsc.VectorSubcoreMesh(core_axis_name="c", subcore_axis_name="s"),
        compiler_params=pltpu.CompilerParams(use_tc_tiling_on_sc=False,
                                             needs_layout_passes=False),
        out_type=jax.ShapeDtypeStruct((nb, num_bags), jnp.float32),
        scratch_types=[
            pltpu.VMEM((items_per_worker,), jnp.int32),
            pltpu.VMEM((16, num_bags + 16), jnp.float32),
            pltpu.VMEM((num_bags,), jnp.float32),
            pltpu.VMEM((num_bags,), jnp.float32),
            pltpu.VMEM_SHARED((8, num_bags), jnp.float32),
        ],
    )(flat_nearest)
    return hist


# speedup vs baseline: 1.8242x; 1.1003x over previous
import functools

import jax
import jax.numpy as jnp
from jax import lax
from jax.experimental import pallas as pl
from jax.experimental.pallas import tpu as pltpu
from jax.experimental.pallas import tpu_sc as plsc


def _assign_kernel(feat_ref, maskf_ref, cent_ref, near_ref, hcn_ref, *,
                   blk, num_bags, strip):
    k = pl.program_id(0)

    @pl.when(k == 0)
    def _hcn():
        cent = cent_ref[...]
        hcn_ref[...] = 0.5 * jnp.sum(cent * cent, axis=1, keepdims=True)

    feat = feat_ref[0]
    nstrips = num_bags // strip
    slabs = strip // 8
    mv = jnp.full((8, blk), -jnp.inf, jnp.float32)
    iv = jnp.zeros((8, blk), jnp.int32)
    u_iota = jax.lax.broadcasted_iota(jnp.int32, (8, blk), 0)
    for s in range(nstrips):
        cs = cent_ref[s * strip:(s + 1) * strip, :]
        sc = jax.lax.dot_general(
            cs, feat, (((1,), (1,)), ((), ())),
            preferred_element_type=jnp.float32)
        d3 = (sc - hcn_ref[s * strip:(s + 1) * strip, :]
              ).reshape(slabs, 8, blk)
        m = jnp.max(d3, axis=0)
        a = jnp.argmax(d3, axis=0).astype(jnp.int32)
        cand = (s * strip + u_iota) + a * 8
        better = m > mv
        mv = jnp.where(better, m, mv)
        iv = jnp.where(better, cand, iv)
    mm = jnp.max(mv, axis=0)
    nearest = jnp.min(jnp.where(mv == mm[None, :], iv, 2 * num_bags),
                      axis=0).astype(jnp.int32)
    masked = maskf_ref[0, 0] > 0.5
    nearest = jnp.where(masked, num_bags, nearest)
    near_ref[...] = nearest[None, None, :]


def _histogram_sc(near_hbm, out_hbm, idx_v, hrows, hv, pv, shared):
    c = lax.axis_index("c")
    s = lax.axis_index("s")
    b_local = s % 8
    half = s // 8
    batch = c * 8 + b_local
    nbins = hrows.shape[1]
    items = idx_v.shape[0]
    base = batch * (2 * items) + half * items

    pltpu.sync_copy(near_hbm.at[pl.ds(base, items)], idx_v)

    zeros16 = jnp.zeros((16,), jnp.float32)
    ones16 = jnp.ones((16,), jnp.float32)
    iota16 = lax.iota(jnp.int32, 16)

    def _zero_body(k, _):
        for r in range(16):
            hrows[r, pl.ds(k * 16, 16)] = zeros16
        return 0

    lax.fori_loop(0, nbins // 16, _zero_body, 0)

    def _acc_body(j, _):
        idxs = idx_v[pl.ds(j * 16, 16)]
        plsc.addupdate_scatter(hrows, [iota16, idxs], ones16)
        return 0

    lax.fori_loop(0, items // 16, _acc_body, 0)

    nred = hv.shape[0] // 16
    for k in range(nred):
        acc = hrows[0, pl.ds(k * 16, 16)]
        for r in range(1, 16):
            acc = acc + hrows[r, pl.ds(k * 16, 16)]
        hv[pl.ds(k * 16, 16)] = acc

    @pl.when(half == 1)
    def _publish():
        pltpu.sync_copy(hv, shared.at[b_local])

    plsc.subcore_barrier()

    @pl.when(half == 0)
    def _merge():
        pltpu.sync_copy(shared.at[b_local], pv)
        for k in range(nred):
            hv[pl.ds(k * 16, 16)] = (hv[pl.ds(k * 16, 16)]
                                     + pv[pl.ds(k * 16, 16)])
        pltpu.sync_copy(hv, out_hbm.at[batch])


def kernel(features, mask, centroids):
    nb, nc, d = features.shape
    num_bags = centroids.shape[0]
    blk = 2048
    num_blk = nc // blk
    maskf = mask.astype(jnp.float32).reshape(nb * num_blk, 1, blk)

    nblocks = nb * num_blk
    feats3 = features.reshape(nblocks, blk, d)
    nearest = pl.pallas_call(
        functools.partial(_assign_kernel, blk=blk, num_bags=num_bags,
                          strip=256),
        grid=(nblocks,),
        in_specs=[
            pl.BlockSpec((1, blk, d), lambda k: (k, 0, 0)),
            pl.BlockSpec((1, 1, blk), lambda k: (k, 0, 0)),
            pl.BlockSpec((num_bags, d), lambda k: (0, 0)),
        ],
        out_specs=pl.BlockSpec((1, 1, blk), lambda k: (k, 0, 0)),
        out_shape=jax.ShapeDtypeStruct((nblocks, 1, blk), jnp.int32),
        scratch_shapes=[pltpu.VMEM((num_bags, 1), jnp.float32)],
        compiler_params=pltpu.CompilerParams(
            dimension_semantics=("arbitrary",)),
    )(feats3, maskf, centroids)

    flat_nearest = nearest.reshape(nb * nc)
    items_per_worker = (nb * nc) // 32

    hist = pl.kernel(
        _histogram_sc,
        mesh=plsc.VectorSubcoreMesh(core_axis_name="c", subcore_axis_name="s"),
        compiler_params=pltpu.CompilerParams(use_tc_tiling_on_sc=False,
                                             needs_layout_passes=False),
        out_type=jax.ShapeDtypeStruct((nb, num_bags), jnp.float32),
        scratch_types=[
            pltpu.VMEM((items_per_worker,), jnp.int32),
            pltpu.VMEM((16, num_bags + 16), jnp.float32),
            pltpu.VMEM((num_bags,), jnp.float32),
            pltpu.VMEM((num_bags,), jnp.float32),
            pltpu.VMEM_SHARED((8, num_bags), jnp.float32),
        ],
    )(flat_nearest)
    return hist


# speedup vs baseline: 1.9158x; 1.0502x over previous
import functools

import jax
import jax.numpy as jnp
from jax import lax
from jax.experimental import pallas as pl
from jax.experimental.pallas import tpu as pltpu
from jax.experimental.pallas import tpu_sc as plsc


def _assign_kernel(feat_ref, maskf_ref, cent_ref, near_ref, hcn_ref, *,
                   blk, num_bags, strip):
    k = pl.program_id(0)

    @pl.when(k == 0)
    def _hcn():
        cent = cent_ref[...]
        hcn_ref[...] = 0.5 * jnp.sum(cent * cent, axis=1, keepdims=True)

    feat = feat_ref[0]
    nstrips = num_bags // strip
    slabs = strip // 8
    mv = jnp.full((8, blk), -jnp.inf, jnp.float32)
    iv = jnp.zeros((8, blk), jnp.int32)
    u_iota = jax.lax.broadcasted_iota(jnp.int32, (8, blk), 0)
    for s in range(nstrips):
        cs = cent_ref[s * strip:(s + 1) * strip, :]
        sc = jax.lax.dot_general(
            cs, feat, (((1,), (1,)), ((), ())),
            preferred_element_type=jnp.float32)
        d3 = (sc - hcn_ref[s * strip:(s + 1) * strip, :]
              ).reshape(slabs, 8, blk)
        pairs = [(d3[g], None) for g in range(slabs)]
        gidx = list(range(slabs))
        while len(pairs) > 1:
            nxt, nidx = [], []
            for t in range(0, len(pairs), 2):
                (va, ia), (vb, ib) = pairs[t], pairs[t + 1]
                ga, gb = gidx[t], gidx[t + 1]
                bwin = vb > va
                v = jnp.where(bwin, vb, va)
                if ia is None and ib is None:
                    idx = jnp.where(bwin, gb, ga)
                else:
                    ia2 = ia if ia is not None else jnp.full(
                        (8, blk), ga, jnp.int32)
                    ib2 = ib if ib is not None else jnp.full(
                        (8, blk), gb, jnp.int32)
                    idx = jnp.where(bwin, ib2, ia2)
                nxt.append((v, idx))
                nidx.append(ga)
            pairs, gidx = nxt, nidx
        m, a = pairs[0]
        cand = (s * strip + u_iota) + a * 8
        better = m > mv
        mv = jnp.where(better, m, mv)
        iv = jnp.where(better, cand, iv)
    mm = jnp.max(mv, axis=0)
    nearest = jnp.min(jnp.where(mv == mm[None, :], iv, 2 * num_bags),
                      axis=0).astype(jnp.int32)
    masked = maskf_ref[0, 0] > 0.5
    nearest = jnp.where(masked, num_bags, nearest)
    near_ref[...] = nearest[None, None, :]


def _histogram_sc(near_hbm, out_hbm, idx_v, hrows, hv, pv, shared):
    c = lax.axis_index("c")
    s = lax.axis_index("s")
    b_local = s % 8
    half = s // 8
    batch = c * 8 + b_local
    nbins = hrows.shape[1]
    items = idx_v.shape[0]
    base = batch * (2 * items) + half * items

    pltpu.sync_copy(near_hbm.at[pl.ds(base, items)], idx_v)

    zeros16 = jnp.zeros((16,), jnp.float32)
    ones16 = jnp.ones((16,), jnp.float32)
    iota16 = lax.iota(jnp.int32, 16)

    def _zero_body(k, _):
        for r in range(16):
            hrows[r, pl.ds(k * 16, 16)] = zeros16
        return 0

    lax.fori_loop(0, nbins // 16, _zero_body, 0)

    def _acc_body(j, _):
        idxs = idx_v[pl.ds(j * 16, 16)]
        plsc.addupdate_scatter(hrows, [iota16, idxs], ones16)
        return 0

    lax.fori_loop(0, items // 16, _acc_body, 0)

    nred = hv.shape[0] // 16
    for k in range(nred):
        acc = hrows[0, pl.ds(k * 16, 16)]
        for r in range(1, 16):
            acc = acc + hrows[r, pl.ds(k * 16, 16)]
        hv[pl.ds(k * 16, 16)] = acc

    @pl.when(half == 1)
    def _publish():
        pltpu.sync_copy(hv, shared.at[b_local])

    plsc.subcore_barrier()

    @pl.when(half == 0)
    def _merge():
        pltpu.sync_copy(shared.at[b_local], pv)
        for k in range(nred):
            hv[pl.ds(k * 16, 16)] = (hv[pl.ds(k * 16, 16)]
                                     + pv[pl.ds(k * 16, 16)])
        pltpu.sync_copy(hv, out_hbm.at[batch])


def kernel(features, mask, centroids):
    nb, nc, d = features.shape
    num_bags = centroids.shape[0]
    blk = 4096
    num_blk = nc // blk
    maskf = mask.astype(jnp.float32).reshape(nb * num_blk, 1, blk)

    nblocks = nb * num_blk
    feats3 = features.reshape(nblocks, blk, d)
    nearest = pl.pallas_call(
        functools.partial(_assign_kernel, blk=blk, num_bags=num_bags,
                          strip=256),
        grid=(nblocks,),
        in_specs=[
            pl.BlockSpec((1, blk, d), lambda k: (k, 0, 0)),
            pl.BlockSpec((1, 1, blk), lambda k: (k, 0, 0)),
            pl.BlockSpec((num_bags, d), lambda k: (0, 0)),
        ],
        out_specs=pl.BlockSpec((1, 1, blk), lambda k: (k, 0, 0)),
        out_shape=jax.ShapeDtypeStruct((nblocks, 1, blk), jnp.int32),
        scratch_shapes=[pltpu.VMEM((num_bags, 1), jnp.float32)],
        compiler_params=pltpu.CompilerParams(
            dimension_semantics=("arbitrary",)),
    )(feats3, maskf, centroids)

    flat_nearest = nearest.reshape(nb * nc)
    items_per_worker = (nb * nc) // 32

    hist = pl.kernel(
        _histogram_sc,
        mesh=plsc.VectorSubcoreMesh(core_axis_name="c", subcore_axis_name="s"),
        compiler_params=pltpu.CompilerParams(use_tc_tiling_on_sc=False,
                                             needs_layout_passes=False),
        out_type=jax.ShapeDtypeStruct((nb, num_bags), jnp.float32),
        scratch_types=[
            pltpu.VMEM((items_per_worker,), jnp.int32),
            pltpu.VMEM((16, num_bags + 16), jnp.float32),
            pltpu.VMEM((num_bags,), jnp.float32),
            pltpu.VMEM((num_bags,), jnp.float32),
            pltpu.VMEM_SHARED((8, num_bags), jnp.float32),
        ],
    )(flat_nearest)
    return hist


# speedup vs baseline: 1.9186x; 1.0014x over previous
import functools

import jax
import jax.numpy as jnp
from jax import lax
from jax.experimental import pallas as pl
from jax.experimental.pallas import tpu as pltpu
from jax.experimental.pallas import tpu_sc as plsc


def _argmax_bins(feat, cent_ref, hcn_ref, *, num_bags, strip):
    sub = feat.shape[0]
    nstrips = num_bags // strip
    slabs = strip // 8
    mv = jnp.full((8, sub), -jnp.inf, jnp.float32)
    iv = jnp.zeros((8, sub), jnp.int32)
    u_iota = jax.lax.broadcasted_iota(jnp.int32, (8, sub), 0)
    for s in range(nstrips):
        cs = cent_ref[s * strip:(s + 1) * strip, :]
        sc = jax.lax.dot_general(
            cs, feat, (((1,), (1,)), ((), ())),
            preferred_element_type=jnp.float32)
        d3 = (sc - hcn_ref[s * strip:(s + 1) * strip, :]
              ).reshape(slabs, 8, sub)
        pairs = [(d3[g], None) for g in range(slabs)]
        gidx = list(range(slabs))
        while len(pairs) > 1:
            nxt, nidx = [], []
            for t in range(0, len(pairs), 2):
                (va, ia), (vb, ib) = pairs[t], pairs[t + 1]
                ga, gb = gidx[t], gidx[t + 1]
                bwin = vb > va
                v = jnp.where(bwin, vb, va)
                if ia is None and ib is None:
                    idx = jnp.where(bwin, gb, ga)
                else:
                    ia2 = ia if ia is not None else jnp.full(
                        (8, sub), ga, jnp.int32)
                    ib2 = ib if ib is not None else jnp.full(
                        (8, sub), gb, jnp.int32)
                    idx = jnp.where(bwin, ib2, ia2)
                nxt.append((v, idx))
                nidx.append(ga)
            pairs, gidx = nxt, nidx
        m, a = pairs[0]
        cand = (s * strip + u_iota) + a * 8
        better = m > mv
        mv = jnp.where(better, m, mv)
        iv = jnp.where(better, cand, iv)
    mm = jnp.max(mv, axis=0)
    return jnp.min(jnp.where(mv == mm[None, :], iv, 2 * num_bags),
                   axis=0).astype(jnp.int32)


def _assign_kernel(*refs, blk, num_bags, strip, nsplit):
    feat_refs = refs[:nsplit]
    maskf_ref, cent_ref, near_ref, hcn_ref = refs[nsplit:nsplit + 4]
    k = pl.program_id(0)
    sub = blk // nsplit

    @pl.when(k == 0)
    def _hcn():
        cent = cent_ref[...]
        hcn_ref[...] = 0.5 * jnp.sum(cent * cent, axis=1, keepdims=True)

    for h in range(nsplit):
        feat = feat_refs[h][0, 0]
        nearest = _argmax_bins(feat, cent_ref, hcn_ref,
                               num_bags=num_bags, strip=strip)
        masked = maskf_ref[0, 0, h * sub:(h + 1) * sub] > 0.5
        nearest = jnp.where(masked, num_bags, nearest)
        near_ref[0, 0, h * sub:(h + 1) * sub] = nearest


def _histogram_sc(near_hbm, out_hbm, idx_v, hrows, hv, pv, shared):
    c = lax.axis_index("c")
    s = lax.axis_index("s")
    b_local = s % 8
    half = s // 8
    batch = c * 8 + b_local
    nbins = hrows.shape[1]
    items = idx_v.shape[0]
    base = batch * (2 * items) + half * items

    pltpu.sync_copy(near_hbm.at[pl.ds(base, items)], idx_v)

    zeros16 = jnp.zeros((16,), jnp.float32)
    ones16 = jnp.ones((16,), jnp.float32)
    iota16 = lax.iota(jnp.int32, 16)

    def _zero_body(k, _):
        for r in range(16):
            hrows[r, pl.ds(k * 16, 16)] = zeros16
        return 0

    lax.fori_loop(0, nbins // 16, _zero_body, 0)

    def _acc_body(j, _):
        idxs = idx_v[pl.ds(j * 16, 16)]
        plsc.addupdate_scatter(hrows, [iota16, idxs], ones16)
        return 0

    lax.fori_loop(0, items // 16, _acc_body, 0)

    nred = hv.shape[0] // 16
    for k in range(nred):
        acc = hrows[0, pl.ds(k * 16, 16)]
        for r in range(1, 16):
            acc = acc + hrows[r, pl.ds(k * 16, 16)]
        hv[pl.ds(k * 16, 16)] = acc

    @pl.when(half == 1)
    def _publish():
        pltpu.sync_copy(hv, shared.at[b_local])

    plsc.subcore_barrier()

    @pl.when(half == 0)
    def _merge():
        pltpu.sync_copy(shared.at[b_local], pv)
        for k in range(nred):
            hv[pl.ds(k * 16, 16)] = (hv[pl.ds(k * 16, 16)]
                                     + pv[pl.ds(k * 16, 16)])
        pltpu.sync_copy(hv, out_hbm.at[batch])


def kernel(features, mask, centroids):
    nb, nc, d = features.shape
    num_bags = centroids.shape[0]
    blk = 4096
    num_blk = nc // blk
    maskf = mask.astype(jnp.float32).reshape(nb * num_blk, 1, blk)

    nblocks = nb * num_blk
    nsplit = 2
    sub = blk // nsplit
    feats4 = features.reshape(nblocks, nsplit, sub, d)
    feat_specs = [
        pl.BlockSpec((1, 1, sub, d), lambda k, hh=h: (k, hh, 0, 0))
        for h in range(nsplit)
    ]
    nearest = pl.pallas_call(
        functools.partial(_assign_kernel, blk=blk, num_bags=num_bags,
                          strip=256, nsplit=nsplit),
        grid=(nblocks,),
        in_specs=feat_specs + [
            pl.BlockSpec((1, 1, blk), lambda k: (k, 0, 0)),
            pl.BlockSpec((num_bags, d), lambda k: (0, 0)),
        ],
        out_specs=pl.BlockSpec((1, 1, blk), lambda k: (k, 0, 0)),
        out_shape=jax.ShapeDtypeStruct((nblocks, 1, blk), jnp.int32),
        scratch_shapes=[pltpu.VMEM((num_bags, 1), jnp.float32)],
        compiler_params=pltpu.CompilerParams(
            dimension_semantics=("arbitrary",)),
    )(*([feats4] * nsplit), maskf, centroids)

    flat_nearest = nearest.reshape(nb * nc)
    items_per_worker = (nb * nc) // 32

    hist = pl.kernel(
        _histogram_sc,
        mesh=plsc.VectorSubcoreMesh(core_axis_name="c", subcore_axis_name="s"),
        compiler_params=pltpu.CompilerParams(use_tc_tiling_on_sc=False,
                                             needs_layout_passes=False),
        out_type=jax.ShapeDtypeStruct((nb, num_bags), jnp.float32),
        scratch_types=[
            pltpu.VMEM((items_per_worker,), jnp.int32),
            pltpu.VMEM((16, num_bags + 16), jnp.float32),
            pltpu.VMEM((num_bags,), jnp.float32),
            pltpu.VMEM((num_bags,), jnp.float32),
            pltpu.VMEM_SHARED((8, num_bags), jnp.float32),
        ],
    )(flat_nearest)
    return hist
